# Initial kernel scaffold; baseline (speedup 1.0000x reference)
import jax
import jax.numpy as jnp
from jax.experimental import pallas as pl


def kernel(queries, W_q, b_q, keys, w_down_embed, w_up_embed):
    raise NotImplementedError("write your pallas kernel here")


# speedup vs baseline: 10.6523x; 10.6523x over previous
import functools

import jax
import jax.numpy as jnp
from jax.experimental import pallas as pl
from jax.experimental.pallas import tpu as pltpu

D_MODEL = 1024
N_HEADS = 8
D_KEYS = 256
HALF = D_KEYS // 2
N_EXPERTS = 64
N_ROWS = N_EXPERTS * N_EXPERTS
TOP_K = 2
NEG = -1e30


def _top2(s, iota_e):
    v1 = jnp.max(s, axis=1, keepdims=True)
    i1 = jnp.min(jnp.where(s == v1, iota_e, N_EXPERTS), axis=1, keepdims=True)
    s_m = jnp.where(iota_e == i1, NEG, s)
    v2 = jnp.max(s_m, axis=1, keepdims=True)
    i2 = jnp.min(jnp.where(s_m == v2, iota_e, N_EXPERTS), axis=1, keepdims=True)
    return v1, i1, v2, i2


def _routing_body(q_ref, wq_ref, bq_ref, keys_ref, idx_ref, gate_ref):
    q = q_ref[...]
    qh = jax.lax.dot_general(q, wq_ref[...], (((1,), (1,)), ((), ())),
                             preferred_element_type=jnp.float32) + bq_ref[...]
    blk = q.shape[0]
    iota_e = jax.lax.broadcasted_iota(jnp.int32, (blk, N_EXPERTS), 1)
    idx_cols, gate_cols = [], []
    for h in range(N_HEADS):
        q1 = qh[:, h * D_KEYS:h * D_KEYS + HALF]
        q2 = qh[:, h * D_KEYS + HALF:(h + 1) * D_KEYS]
        s1 = jax.lax.dot_general(q1, keys_ref[2 * h], (((1,), (1,)), ((), ())),
                                 preferred_element_type=jnp.float32)
        s2 = jax.lax.dot_general(q2, keys_ref[2 * h + 1], (((1,), (1,)), ((), ())),
                                 preferred_element_type=jnp.float32)
        v1a, i1a, v1b, i1b = _top2(s1, iota_e)
        v2a, i2a, v2b, i2b = _top2(s2, iota_e)
        cv = [v1a + v2a, v1a + v2b, v1b + v2a, v1b + v2b]
        ci = [i1a * N_EXPERTS + i2a, i1a * N_EXPERTS + i2b,
              i1b * N_EXPERTS + i2a, i1b * N_EXPERTS + i2b]
        bv, bi, bp = cv[0], ci[0], jnp.zeros_like(ci[0])
        for j in range(1, 4):
            cond = cv[j] > bv
            bv = jnp.where(cond, cv[j], bv)
            bi = jnp.where(cond, ci[j], bi)
            bp = jnp.where(cond, j, bp)
        sv = jnp.full_like(bv, NEG)
        si = jnp.zeros_like(bi)
        for j in range(4):
            cond = (cv[j] > sv) & (bp != j)
            sv = jnp.where(cond, cv[j], sv)
            si = jnp.where(cond, ci[j], si)
        e = jnp.exp(sv - bv)
        g0 = 1.0 / (1.0 + e)
        g1 = e / (1.0 + e)
        idx_cols += [bi, si]
        gate_cols += [g0, g1]
    idx_ref[...] = jnp.concatenate(idx_cols, axis=1)
    gate_ref[...] = jnp.concatenate(gate_cols, axis=1)


def _combine_body(q_ref, wd_ref, wu_ref, idx_ref, gate_ref, out_ref):
    q = q_ref[...]
    blk = q.shape[0]
    hfull = jax.lax.dot_general(q, wd_ref[...], (((1,), (1,)), ((), ())),
                                preferred_element_type=jnp.float32)
    iota_r = jax.lax.broadcasted_iota(jnp.int32, (blk, N_ROWS), 1)
    s_acc = jnp.zeros((blk, N_ROWS), jnp.float32)
    for hk in range(N_HEADS * TOP_K):
        m = iota_r == idx_ref[:, hk:hk + 1]
        hcol = jnp.sum(jnp.where(m, hfull, 0.0), axis=1, keepdims=True)
        val = jax.nn.gelu(hcol) * gate_ref[:, hk:hk + 1]
        s_acc = s_acc + jnp.where(m, val, 0.0)
    out_ref[...] = jax.lax.dot_general(s_acc, wu_ref[...], (((1,), (0,)), ((), ())),
                                       preferred_element_type=jnp.float32)


def _moe(queries, W_q, b_q, keys, w_down_embed, w_up_embed, interpret=False):
    B, T, D = queries.shape
    NT = B * T
    q_flat = queries.reshape(NT, D)
    keys_r = keys.reshape(2 * N_HEADS, N_EXPERTS, HALF)
    bq_r = b_q.reshape(1, N_HEADS * D_KEYS)

    blk = 512
    grid = (NT // blk,)

    idx, gates = pl.pallas_call(
        _routing_body,
        grid=grid,
        in_specs=[
            pl.BlockSpec((blk, D), lambda i: (i, 0)),
            pl.BlockSpec((N_HEADS * D_KEYS, D), lambda i: (0, 0)),
            pl.BlockSpec((1, N_HEADS * D_KEYS), lambda i: (0, 0)),
            pl.BlockSpec((2 * N_HEADS, N_EXPERTS, HALF), lambda i: (0, 0, 0)),
        ],
        out_specs=[
            pl.BlockSpec((blk, N_HEADS * TOP_K), lambda i: (i, 0)),
            pl.BlockSpec((blk, N_HEADS * TOP_K), lambda i: (i, 0)),
        ],
        out_shape=[
            jax.ShapeDtypeStruct((NT, N_HEADS * TOP_K), jnp.int32),
            jax.ShapeDtypeStruct((NT, N_HEADS * TOP_K), jnp.float32),
        ],
        interpret=interpret,
    )(q_flat, W_q, bq_r, keys_r)

    cblk = 256
    out = pl.pallas_call(
        _combine_body,
        grid=(NT // cblk,),
        in_specs=[
            pl.BlockSpec((cblk, D), lambda i: (i, 0)),
            pl.BlockSpec((N_ROWS, D), lambda i: (0, 0)),
            pl.BlockSpec((N_ROWS, D), lambda i: (0, 0)),
            pl.BlockSpec((cblk, N_HEADS * TOP_K), lambda i: (i, 0)),
            pl.BlockSpec((cblk, N_HEADS * TOP_K), lambda i: (i, 0)),
        ],
        out_specs=pl.BlockSpec((cblk, D), lambda i: (i, 0)),
        out_shape=jax.ShapeDtypeStruct((NT, D), jnp.float32),
        interpret=interpret,
    )(q_flat, w_down_embed, w_up_embed, idx, gates)

    return out.reshape(B, T, D)


def kernel(queries, W_q, b_q, keys, w_down_embed, w_up_embed):
    return _moe(queries, W_q, b_q, keys, w_down_embed, w_up_embed)
